# baseline (speedup 1.0000x reference)
import jax
import jax.numpy as jnp
from jax.experimental import pallas as pl
from jax.experimental.pallas import tpu as pltpu

_CLIP = 16
_N = 2048
_NOUT = 64
_ROWS = 2 * _CLIP + 1
_FLEN = 2 * _N - 1
_G = 16
_RPS = _N // _G
_DEPTH = 8


def _rpe_kernel(e_ref, o_ref, f_ref, sem):
    lo = jnp.broadcast_to(e_ref[0:1, :], (_N - _CLIP - 1, _NOUT))
    hi = jnp.broadcast_to(e_ref[_ROWS - 1:_ROWS, :], (_N - _CLIP - 1, _NOUT))
    f_ref[0:_N - _CLIP - 1, :] = lo
    f_ref[_N - _CLIP - 1:_N + _CLIP, :] = e_ref[:, :]
    f_ref[_N + _CLIP:_FLEN, :] = hi

    r0 = pl.program_id(0) * _RPS

    def _copy(r, s):
        return pltpu.make_async_copy(
            f_ref.at[pl.ds(_N - 1 - r, _N), :], o_ref.at[r], sem.at[s])

    def body(j, carry):
        for u in range(_DEPTH):
            r = r0 + j * _DEPTH + u

            @pl.when(j > 0)
            def _():
                _copy(r - _DEPTH, u).wait()

            _copy(r, u).start()
        return carry

    jax.lax.fori_loop(0, _RPS // _DEPTH, body, 0)
    for u in range(_DEPTH):
        _copy(r0 + _RPS - _DEPTH + u, u).wait()


def kernel(encoding_matrix, num_keys, offset):
    del num_keys, offset
    return pl.pallas_call(
        _rpe_kernel,
        grid=(_G,),
        in_specs=[pl.BlockSpec(memory_space=pltpu.MemorySpace.VMEM)],
        out_specs=pl.BlockSpec(memory_space=pltpu.MemorySpace.HBM),
        out_shape=jax.ShapeDtypeStruct((_N, _N, _NOUT), jnp.float32),
        scratch_shapes=[
            pltpu.VMEM((_FLEN, _NOUT), jnp.float32),
            pltpu.SemaphoreType.DMA((_DEPTH,)),
        ],
        compiler_params=pltpu.CompilerParams(
            dimension_semantics=("parallel",)),
    )(encoding_matrix)
